# baseline (speedup 1.0000x reference)
import jax
import jax.numpy as jnp
from jax.experimental import pallas as pl
from jax.experimental.pallas import tpu as pltpu

_B, _L, _J, _D = 4, 2048, 8, 256
_DX, _TED = 6, 6
_NS = _DX * (_TED - 1)
_JH = 4
_H = _J // _JH


def _emb_kernel(x_ref, y_ref, a_ref, ss_ref, bs_ref, w1s_ref, w0_ref,
                vtb_ref, local_ref, d01_ref, space_ref,
                out_vt_ref, out_sp_ref, base_ref):
    h = pl.program_id(1)

    @pl.when(h == 0)
    def _():
        xb = jnp.nan_to_num(x_ref[0])
        lin = jax.lax.dot(xb, a_ref[...], precision=jax.lax.Precision.HIGHEST,
                          preferred_element_type=jnp.float32)
        xs = jax.lax.dot(xb, ss_ref[...], precision=jax.lax.Precision.HIGHEST,
                         preferred_element_type=jnp.float32) + bs_ref[...]
        sin_part = jax.lax.dot(jnp.sin(xs), w1s_ref[...],
                               precision=jax.lax.Precision.HIGHEST,
                               preferred_element_type=jnp.float32)
        base_ref[...] = local_ref[...] + lin + sin_part + vtb_ref[...]

    base = base_ref[...]
    for jj in range(_JH):
        ycol = y_ref[jj]
        nanf = jnp.isnan(ycol).astype(jnp.float32)
        yc = jnp.where(nanf > 0, jnp.float32(0), ycol)
        out_vt_ref[0, jj * _L:(jj + 1) * _L, :] = (
            base + yc * w0_ref[...] + nanf * d01_ref[...])
        row = space_ref[pl.ds(h * _JH + jj, 1), :]
        out_sp_ref[0, jj * _L:(jj + 1) * _L, :] = jnp.broadcast_to(row, (_L, _D))


def kernel(y, x, t2v_w, t2v_b, local_table, vt_W, vt_b, given_table, space_table):
    w1 = vt_W[1:]
    a_mat = t2v_w[:, 0:1] * w1[0::_TED]
    c_lin = t2v_b[:, 0] @ w1[0::_TED]
    ss = (jnp.eye(_DX, dtype=jnp.float32)[:, :, None]
          * t2v_w[:, None, 1:]).reshape(_DX, _NS)
    bs = t2v_b[:, 1:].reshape(1, _NS)
    sin_rows = (jnp.arange(_DX * _TED).reshape(_DX, _TED)[:, 1:]).reshape(-1)
    w1s = w1[sin_rows]
    w0 = vt_W[0:1]
    vtb2 = (vt_b + c_lin + given_table[1]).reshape(1, _D)
    d01 = (given_table[0] - given_table[1]).reshape(1, _D)
    y_t = jnp.transpose(y, (0, 2, 1)).reshape(_B * _J, _L, 1)

    grid = (_B, _H)
    out_vt, out_sp = pl.pallas_call(
        _emb_kernel,
        grid=grid,
        in_specs=[
            pl.BlockSpec((1, _L, _DX), lambda b, h: (b, 0, 0)),
            pl.BlockSpec((_JH, _L, 1), lambda b, h: (b * _H + h, 0, 0)),
            pl.BlockSpec((_DX, _D), lambda b, h: (0, 0)),
            pl.BlockSpec((_DX, _NS), lambda b, h: (0, 0)),
            pl.BlockSpec((1, _NS), lambda b, h: (0, 0)),
            pl.BlockSpec((_NS, _D), lambda b, h: (0, 0)),
            pl.BlockSpec((1, _D), lambda b, h: (0, 0)),
            pl.BlockSpec((1, _D), lambda b, h: (0, 0)),
            pl.BlockSpec((_L, _D), lambda b, h: (0, 0)),
            pl.BlockSpec((1, _D), lambda b, h: (0, 0)),
            pl.BlockSpec((_J, _D), lambda b, h: (0, 0)),
        ],
        out_specs=[
            pl.BlockSpec((1, _JH * _L, _D), lambda b, h: (b, h, 0)),
            pl.BlockSpec((1, _JH * _L, _D), lambda b, h: (b, h, 0)),
        ],
        out_shape=[
            jax.ShapeDtypeStruct((_B, _J * _L, _D), jnp.float32),
            jax.ShapeDtypeStruct((_B, _J * _L, _D), jnp.float32),
        ],
        scratch_shapes=[pltpu.VMEM((_L, _D), jnp.float32)],
        compiler_params=pltpu.CompilerParams(
            dimension_semantics=("parallel", "arbitrary")),
    )(x, y_t, a_mat, ss, bs, w1s, w0, vtb2,
      local_table[:_L], d01, space_table)

    var_idx = jnp.broadcast_to(
        jnp.repeat(jnp.arange(_J, dtype=jnp.int32), _L)[None, :],
        (_B, _J * _L))
    return (out_vt, out_sp, var_idx)
